# baseline (speedup 1.0000x reference)
import functools

import jax
import jax.numpy as jnp
from jax import lax
from jax.experimental import pallas as pl
from jax.experimental.pallas import tpu as pltpu
from jax.experimental.pallas import tpu_sc as plsc

K = 8
BT = 512
NW = 32
L = 16


def _mlp_body(x_ref, W1_ref, b1_ref, W2_ref, b2_ref, logits_ref):
    x = x_ref[...]
    h = lax.dot_general(x, W1_ref[...], (((1,), (1,)), ((), ())),
                        preferred_element_type=jnp.float32)
    h = jnp.maximum(h + b1_ref[...], 0.0)
    logits = lax.dot_general(h, W2_ref[...], (((1,), (1,)), ((), ())),
                             preferred_element_type=jnp.float32)
    logits_ref[...] = logits + b2_ref[...]


def _mlp_logits(x, W1, b1, W2, b2):
    T, D = x.shape
    H = W1.shape[0]
    E = W2.shape[0]
    return pl.pallas_call(
        _mlp_body,
        grid=(T // BT,),
        in_specs=[
            pl.BlockSpec((BT, D), lambda i: (i, 0)),
            pl.BlockSpec((H, D), lambda i: (0, 0)),
            pl.BlockSpec((1, H), lambda i: (0, 0)),
            pl.BlockSpec((E, H), lambda i: (0, 0)),
            pl.BlockSpec((1, E), lambda i: (0, 0)),
        ],
        out_specs=pl.BlockSpec((BT, E), lambda i: (i, 0)),
        out_shape=jax.ShapeDtypeStruct((T, E), jnp.float32),
    )(x, W1, b1.reshape(1, H), W2, b2.reshape(1, E))


def _make_router_sc(T, E):
    RW = T // NW
    CH = 256
    mesh = plsc.VectorSubcoreMesh(core_axis_name="c", subcore_axis_name="s")

    @functools.partial(
        pl.kernel,
        mesh=mesh,
        out_type=[
            jax.ShapeDtypeStruct((T, E), jnp.float32),
            jax.ShapeDtypeStruct((T, K), jnp.int32),
        ],
        scratch_types=[
            pltpu.VMEM((CH, E), jnp.float32),
            pltpu.VMEM((CH, E), jnp.float32),
            pltpu.VMEM((CH, K), jnp.int32),
        ],
        compiler_params=pltpu.CompilerParams(needs_layout_passes=False),
    )
    def route(logits_hbm, probs_hbm, idx_hbm, lg_v, pr_v, ix_v):
        c = lax.axis_index("c")
        s = lax.axis_index("s")
        wid = s * 2 + c
        base = wid * RW

        iota = lax.iota(jnp.int32, L)
        lane_lt8 = iota < K
        neg_inf = jnp.float32(-jnp.inf)
        zeros = jnp.zeros((L,), jnp.float32)

        def merge(av, ai, bv, bi):
            rbv = lax.rev(bv, (0,))
            rbi = lax.rev(bi, (0,))
            take_a = av >= rbv
            cv = jnp.where(take_a, av, rbv)
            ci = jnp.where(take_a, ai, rbi)
            return plsc.sort_key_val(cv, ci, descending=True)

        def row_body(r):
            sv = []
            si = []
            for j in range(4):
                vj, ij = plsc.sort_key_val(
                    lg_v[r, pl.ds(L * j, L)], iota + L * j, descending=True)
                sv.append(vj)
                si.append(ij)
            v01, i01 = merge(sv[0], si[0], sv[1], si[1])
            v23, i23 = merge(sv[2], si[2], sv[3], si[3])
            fv, fi = merge(v01, i01, v23, i23)
            m = jnp.max(jnp.where(lane_lt8, fv, neg_inf))
            e = jnp.where(lane_lt8, jnp.exp(fv - m), 0.0)
            p = e / jnp.sum(e)
            rowvec = jnp.full((L,), r, jnp.int32)
            for j in range(4):
                pr_v[r, pl.ds(L * j, L)] = zeros
            plsc.store_scatter(pr_v, [rowvec, fi], p, mask=lane_lt8)
            plsc.store_scatter(ix_v, [rowvec, iota], fi, mask=lane_lt8)

        def chunk_body(ci, carry):
            rbase = base + ci * CH
            pltpu.sync_copy(logits_hbm.at[pl.ds(rbase, CH)], lg_v)
            plsc.parallel_loop(0, CH, unroll=8)(row_body)
            pltpu.sync_copy(pr_v, probs_hbm.at[pl.ds(rbase, CH)])
            pltpu.sync_copy(ix_v, idx_hbm.at[pl.ds(rbase, CH)])
            return carry

        lax.fori_loop(0, RW // CH, chunk_body, 0)

    return route


def kernel(x, W1, b1, W2, b2):
    T = x.shape[0]
    E = W2.shape[0]
    logits = _mlp_logits(x, W1, b1, W2, b2)
    probs, idx = _make_router_sc(T, E)(logits)
    return (probs, idx)


# speedup vs baseline: 1.0135x; 1.0135x over previous
import functools

import jax
import jax.numpy as jnp
from jax import lax
from jax.experimental import pallas as pl
from jax.experimental.pallas import tpu as pltpu
from jax.experimental.pallas import tpu_sc as plsc

K = 8
BT = 512
NW = 32
L = 16


def _mlp_body(x_ref, W1_ref, b1_ref, W2_ref, b2_ref, logits_ref):
    x = x_ref[...]
    h = lax.dot_general(x, W1_ref[...], (((1,), (1,)), ((), ())),
                        preferred_element_type=jnp.float32)
    h = jnp.maximum(h + b1_ref[...], 0.0)
    logits = lax.dot_general(h, W2_ref[...], (((1,), (1,)), ((), ())),
                             preferred_element_type=jnp.float32)
    logits_ref[...] = logits + b2_ref[...]


def _mlp_logits(x, W1, b1, W2, b2):
    T, D = x.shape
    H = W1.shape[0]
    E = W2.shape[0]
    return pl.pallas_call(
        _mlp_body,
        grid=(T // BT,),
        in_specs=[
            pl.BlockSpec((BT, D), lambda i: (i, 0)),
            pl.BlockSpec((H, D), lambda i: (0, 0)),
            pl.BlockSpec((1, H), lambda i: (0, 0)),
            pl.BlockSpec((E, H), lambda i: (0, 0)),
            pl.BlockSpec((1, E), lambda i: (0, 0)),
        ],
        out_specs=pl.BlockSpec((BT, E), lambda i: (i, 0)),
        out_shape=jax.ShapeDtypeStruct((T, E), jnp.float32),
    )(x, W1, b1.reshape(1, H), W2, b2.reshape(1, E))


def _make_router_sc(T, E):
    RW = T // NW
    CH = 128
    mesh = plsc.VectorSubcoreMesh(core_axis_name="c", subcore_axis_name="s")

    @functools.partial(
        pl.kernel,
        mesh=mesh,
        out_type=[
            jax.ShapeDtypeStruct((T, E), jnp.float32),
            jax.ShapeDtypeStruct((T, K), jnp.int32),
        ],
        scratch_types=[
            pltpu.VMEM((CH, E), jnp.float32),
            pltpu.VMEM((CH, E), jnp.float32),
            pltpu.VMEM((CH, K), jnp.int32),
        ],
        compiler_params=pltpu.CompilerParams(needs_layout_passes=False),
    )
    def route(logits_hbm, probs_hbm, idx_hbm, lg_v, pr_v, ix_v):
        c = lax.axis_index("c")
        s = lax.axis_index("s")
        wid = s * 2 + c
        base = wid * RW

        iota = lax.iota(jnp.int32, L)
        lane_lt8 = iota < K
        neg_inf = jnp.float32(-jnp.inf)
        zeros = jnp.zeros((L,), jnp.float32)

        def merge(av, ai, bv, bi):
            rbv = lax.rev(bv, (0,))
            rbi = lax.rev(bi, (0,))
            take_a = av >= rbv
            cv = jnp.where(take_a, av, rbv)
            ci = jnp.where(take_a, ai, rbi)
            return plsc.sort_key_val(cv, ci, descending=True)

        def row_body(r):
            sv = []
            si = []
            for j in range(4):
                vj, ij = plsc.sort_key_val(
                    lg_v[r, pl.ds(L * j, L)], iota + L * j, descending=True)
                sv.append(vj)
                si.append(ij)
            v01, i01 = merge(sv[0], si[0], sv[1], si[1])
            v23, i23 = merge(sv[2], si[2], sv[3], si[3])
            fv, fi = merge(v01, i01, v23, i23)
            m = jnp.max(jnp.where(lane_lt8, fv, neg_inf))
            e = jnp.where(lane_lt8, jnp.exp(fv - m), 0.0)
            p = e / jnp.sum(e)
            rowvec = jnp.full((L,), r, jnp.int32)
            for j in range(4):
                pr_v[r, pl.ds(L * j, L)] = zeros
            plsc.store_scatter(pr_v, [rowvec, fi], p, mask=lane_lt8)
            plsc.store_scatter(ix_v, [rowvec, iota], fi, mask=lane_lt8)

        def chunk_body(ci, carry):
            rbase = base + ci * CH
            pltpu.sync_copy(logits_hbm.at[pl.ds(rbase, CH)], lg_v)
            plsc.parallel_loop(0, CH, unroll=4)(row_body)
            pltpu.sync_copy(pr_v, probs_hbm.at[pl.ds(rbase, CH)])
            pltpu.sync_copy(ix_v, idx_hbm.at[pl.ds(rbase, CH)])
            return carry

        lax.fori_loop(0, RW // CH, chunk_body, 0)

    return route


def kernel(x, W1, b1, W2, b2):
    T = x.shape[0]
    E = W2.shape[0]
    logits = _mlp_logits(x, W1, b1, W2, b2)
    probs, idx = _make_router_sc(T, E)(logits)
    return (probs, idx)
